# Initial kernel scaffold; baseline (speedup 1.0000x reference)
import jax
import jax.numpy as jnp
from jax.experimental import pallas as pl


def kernel(indices, num_valid_coordinates, padded_features):
    raise NotImplementedError("write your pallas kernel here")


# speedup vs baseline: 10.4794x; 10.4794x over previous
import functools

import jax
import jax.numpy as jnp
from jax import lax
from jax.experimental import pallas as pl
from jax.experimental.pallas import tpu as pltpu
from jax.experimental.pallas import tpu_sc as plsc

_B = 16
_M = 8192
_H = 512
_W = 512
_NC = 2
_NS = 16
_QROWS = 128


@functools.cache
def _build_scatter_kernel():
    mesh = plsc.VectorSubcoreMesh(core_axis_name="c", subcore_axis_name="s")

    @functools.partial(
        pl.kernel,
        out_type=jax.ShapeDtypeStruct((_B, _H, _W), jnp.float32),
        mesh=mesh,
        scratch_types=[
            pltpu.VMEM((_M,), jnp.int32),
            pltpu.VMEM((_M,), jnp.int32),
            pltpu.VMEM((_M,), jnp.float32),
            pltpu.VMEM((16,), jnp.int32),
            pltpu.VMEM((_QROWS, _W), jnp.float32),
        ],
        compiler_params=pltpu.CompilerParams(needs_layout_passes=False),
    )
    def k(rows_hbm, cols_hbm, vals_hbm, nv_hbm, out_hbm,
          rows_v, cols_v, vals_v, nv_v, slab):
        wid = lax.axis_index("s") * _NC + lax.axis_index("c")
        b = wid // 2
        h = wid % 2
        pltpu.sync_copy(rows_hbm.at[b], rows_v)
        pltpu.sync_copy(cols_hbm.at[b], cols_v)
        pltpu.sync_copy(vals_hbm.at[b], vals_v)
        pltpu.sync_copy(nv_hbm, nv_v)
        lane = lax.iota(jnp.int32, 16)
        n = jnp.max(jnp.where(lane == b, nv_v[...], 0))
        zeros = jnp.zeros((16,), jnp.float32)

        for q in range(2):
            lo = h * (2 * _QROWS) + q * _QROWS

            def zero_row(i, carry):
                for j in range(_W // 16):
                    slab[i, pl.ds(j * 16, 16)] = zeros
                return carry

            lax.fori_loop(0, _QROWS, zero_row, 0)

            def scatter_group(g, carry):
                base = g * 16
                r = rows_v[pl.ds(base, 16)]
                c = cols_v[pl.ds(base, 16)]
                v = vals_v[pl.ds(base, 16)]
                rr = r - lo
                m = ((lane + base) < n) & (rr >= 0) & (rr < _QROWS)
                rr = jnp.where(m, rr, 0)
                cc = jnp.where(m, c, 0)
                plsc.store_scatter(slab, [rr, cc], v, mask=m)
                return carry

            lax.fori_loop(0, _M // 16, scatter_group, 0)
            pltpu.sync_copy(slab, out_hbm.at[b, pl.ds(lo, _QROWS)])

    return k


def kernel(indices, num_valid_coordinates, padded_features):
    rows = indices[..., 0]
    cols = indices[..., 1]
    vals = padded_features[..., 0]
    return _build_scatter_kernel()(rows, cols, vals, num_valid_coordinates)
